# Initial kernel scaffold; baseline (speedup 1.0000x reference)
import jax
import jax.numpy as jnp
from jax.experimental import pallas as pl


def kernel(inputs, coords, targets, input_lengths, W1, b1, W2, b2, fc_w, fc_b):
    raise NotImplementedError("write your pallas kernel here")


# speedup vs baseline: 24.0953x; 24.0953x over previous
import functools

import jax
import jax.numpy as jnp
from jax.experimental import pallas as pl
from jax.experimental.pallas import tpu as pltpu

K = 16
L = 2048
B = 8
RT = 256
NEG_INF = float("-inf")
POS_INF = float("inf")


def _knn_body(len_ref, q_ref, ct_ref, d_ref, idx_ref, dsum_ref, mx_ref, mn_ref):
    b = pl.program_id(0)
    j = pl.program_id(1)
    n = len_ref[b]
    q = q_ref[0]
    qx = q[:, 0:1]
    qy = q[:, 1:2]
    kx = ct_ref[0, 0:1, :]
    ky = ct_ref[0, 1:2, :]
    dx = qx - kx
    dy = qy - ky
    d2 = dx * dx + dy * dy
    col = jax.lax.broadcasted_iota(jnp.int32, (RT, L), 1)
    row = j * RT + jax.lax.broadcasted_iota(jnp.int32, (RT, L), 0)
    d2 = jnp.where(col == row, d2 + 1e10, d2)
    d2 = jnp.where(col < n, d2, POS_INF)

    ds, idxs = [], []
    v = d2
    for _ in range(K):
        m = jnp.min(v, axis=1, keepdims=True)
        am = jnp.min(jnp.where(v == m, col, L), axis=1, keepdims=True)
        v = jnp.where(col == am, POS_INF, v)
        ds.append(jnp.sqrt(m))
        idxs.append(am)
    dmat = jnp.concatenate(ds, axis=1)
    imat = jnp.concatenate(idxs, axis=1)
    d_ref[0] = dmat
    idx_ref[0] = imat
    dsum_ref[0, 0, :] = jnp.sum(dmat, axis=1)

    rv = row[:, 0:1] < n
    t_mx = jnp.max(jnp.where(rv, ds[K - 1], NEG_INF))
    t_mn = jnp.min(jnp.where(rv, ds[0], POS_INF))

    @pl.when((b == 0) & (j == 0))
    def _():
        mx_ref[...] = jnp.full((1, 1), NEG_INF, jnp.float32)
        mn_ref[...] = jnp.full((1, 1), POS_INF, jnp.float32)

    mx_ref[...] = jnp.maximum(mx_ref[...], t_mx)
    mn_ref[...] = jnp.minimum(mn_ref[...], t_mn)


def _run_knn(lengths, coords, coords_t):
    grid_spec = pltpu.PrefetchScalarGridSpec(
        num_scalar_prefetch=1,
        grid=(B, L // RT),
        in_specs=[
            pl.BlockSpec((1, RT, 2), lambda b, j, lens: (b, j, 0)),
            pl.BlockSpec((1, 2, L), lambda b, j, lens: (b, 0, 0)),
        ],
        out_specs=[
            pl.BlockSpec((1, RT, K), lambda b, j, lens: (b, j, 0)),
            pl.BlockSpec((1, RT, K), lambda b, j, lens: (b, j, 0)),
            pl.BlockSpec((1, 1, RT), lambda b, j, lens: (b, 0, j)),
            pl.BlockSpec((1, 1), lambda b, j, lens: (0, 0)),
            pl.BlockSpec((1, 1), lambda b, j, lens: (0, 0)),
        ],
    )
    return pl.pallas_call(
        _knn_body,
        grid_spec=grid_spec,
        out_shape=[
            jax.ShapeDtypeStruct((B, L, K), jnp.float32),
            jax.ShapeDtypeStruct((B, L, K), jnp.int32),
            jax.ShapeDtypeStruct((B, 1, L), jnp.float32),
            jax.ShapeDtypeStruct((1, 1), jnp.float32),
            jax.ShapeDtypeStruct((1, 1), jnp.float32),
        ],
    )(lengths, coords, coords_t)


def _deg_body(len_ref, dsum_ref, mx_ref, mn_ref, dinv_ref, stats_ref):
    mxv = mx_ref[...]
    mnv = mn_ref[...]
    ir = 1.0 / (mxv - mnv)
    liota = jax.lax.broadcasted_iota(jnp.int32, (1, L), 1)
    for b in range(B):
        n = len_ref[b]
        valid = liota < n
        deg = 1.0 + (K * mxv - dsum_ref[b, 0, :][None, :]) * ir
        dinv_ref[b, 0, :] = jnp.where(valid, jax.lax.rsqrt(deg), 0.0)[0, :]
    srow = jax.lax.broadcasted_iota(jnp.int32, (8, 128), 0)
    stats_ref[...] = jnp.where(srow == 0, mxv, jnp.where(srow == 1, ir, 0.0))


def _run_deg(lengths, dsum, mx, mn):
    grid_spec = pltpu.PrefetchScalarGridSpec(
        num_scalar_prefetch=1,
        grid=(1,),
        in_specs=[
            pl.BlockSpec((B, 1, L), lambda i, lens: (0, 0, 0)),
            pl.BlockSpec((1, 1), lambda i, lens: (0, 0)),
            pl.BlockSpec((1, 1), lambda i, lens: (0, 0)),
        ],
        out_specs=[
            pl.BlockSpec((B, 1, L), lambda i, lens: (0, 0, 0)),
            pl.BlockSpec((8, 128), lambda i, lens: (0, 0)),
        ],
    )
    return pl.pallas_call(
        _deg_body,
        grid_spec=grid_spec,
        out_shape=[
            jax.ShapeDtypeStruct((B, 1, L), jnp.float32),
            jax.ShapeDtypeStruct((8, 128), jnp.float32),
        ],
    )(lengths, dsum, mx, mn)


def _head_body(ti_ref, td_ref, dinv_ref, x_ref, stats_ref,
               w1_ref, b1_ref, w2_ref, b2_ref, fcw_ref, out_ref):
    ti = ti_ref[0].astype(jnp.float32)
    td = td_ref[0]
    dv = dinv_ref[0, 0, :]
    xb = x_ref[0]
    mxv = stats_ref[0:1, 0:1]
    ir = stats_ref[1:2, 0:1]

    n0 = ti_ref[0][0:1, :]
    d0 = td[0:1, :]

    colL = jax.lax.broadcasted_iota(jnp.int32, (K, L), 1)
    rowL = jax.lax.broadcasted_iota(jnp.int32, (L, K), 0)
    ohT = (rowL == n0).astype(jnp.float32)
    _gath = functools.partial(
        jax.lax.dot_general,
        dimension_numbers=(((0,), (0,)), ((), ())),
        precision=jax.lax.Precision.HIGHEST,
        preferred_element_type=jnp.float32)
    NBf = _gath(ohT, ti)
    DD = _gath(ohT, td)
    XN = _gath(ohT, xb)
    dinv_n0 = _gath(ohT, dv[:, None])

    NBi = NBf.astype(jnp.int32)
    EW = (mxv - DD) * ir
    aggS = dinv_n0 * dinv_n0 * XN
    for j in range(K):
        ohj = (NBi[:, j:j + 1] == colL).astype(jnp.float32)
        Xj = jnp.dot(ohj, xb, precision=jax.lax.Precision.HIGHEST,
                     preferred_element_type=jnp.float32)
        dinv_j = jnp.dot(ohj, dv[:, None],
                         precision=jax.lax.Precision.HIGHEST,
                         preferred_element_type=jnp.float32)
        coeff_j = dinv_n0 * EW[:, j:j + 1] * dinv_j
        aggS = aggS + coeff_j * Xj

    dh = dinv_ref[0, 0:1, 0:1]
    ew0 = (mxv - d0) * ir
    c0 = dh * ew0 * dinv_n0.reshape(1, K)
    xh = xb[0:1, :]
    aggH = dh * dh * xh + jnp.dot(c0, XN,
                                  precision=jax.lax.Precision.HIGHEST,
                                  preferred_element_type=jnp.float32)

    agg1 = jnp.concatenate([aggH, aggS], axis=0)
    h1 = jax.nn.relu(jnp.dot(agg1, w1_ref[...],
                             precision=jax.lax.Precision.HIGHEST,
                             preferred_element_type=jnp.float32) + b1_ref[...])
    agg2 = dh * dh * h1[0:1, :] + jnp.dot(
        c0, h1[1:, :], precision=jax.lax.Precision.HIGHEST,
        preferred_element_type=jnp.float32)
    h2 = jax.nn.relu(jnp.dot(agg2, w2_ref[...],
                             precision=jax.lax.Precision.HIGHEST,
                             preferred_element_type=jnp.float32) + b2_ref[...])
    out_ref[...] = jnp.sum(h2 * fcw_ref[...]).reshape(1, 1, 1)


def _run_head(ti, td, dinv, x, stats, W1, b1, W2, b2, fc_w):
    return pl.pallas_call(
        _head_body,
        grid=(B,),
        in_specs=[
            pl.BlockSpec((1, L, K), lambda b: (b, 0, 0)),
            pl.BlockSpec((1, L, K), lambda b: (b, 0, 0)),
            pl.BlockSpec((1, 1, L), lambda b: (b, 0, 0)),
            pl.BlockSpec((1, L, 128), lambda b: (b, 0, 0)),
            pl.BlockSpec((8, 128), lambda b: (0, 0)),
            pl.BlockSpec((128, 256), lambda b: (0, 0)),
            pl.BlockSpec((1, 256), lambda b: (0, 0)),
            pl.BlockSpec((256, 256), lambda b: (0, 0)),
            pl.BlockSpec((1, 256), lambda b: (0, 0)),
            pl.BlockSpec((1, 256), lambda b: (0, 0)),
        ],
        out_specs=pl.BlockSpec((1, 1, 1), lambda b: (b, 0, 0)),
        out_shape=jax.ShapeDtypeStruct((B, 1, 1), jnp.float32),
    )(ti, td, dinv, x, stats, W1, b1, W2, b2, fc_w)


def kernel(inputs, coords, targets, input_lengths, W1, b1, W2, b2, fc_w, fc_b):
    lengths = input_lengths.astype(jnp.int32)
    coords_t = coords.transpose(0, 2, 1)
    td, ti, dsum, mx, mn = _run_knn(lengths, coords, coords_t)
    dinv, stats = _run_deg(lengths, dsum, mx, mn)
    out = _run_head(ti, td, dinv, inputs, stats,
                    W1, b1.reshape(1, -1), W2, b2.reshape(1, -1),
                    fc_w.reshape(1, -1))
    output_head = out[:, :, 0] + fc_b.reshape(1, 1)
    target_head = targets[:, 0, :]
    return output_head, target_head


# speedup vs baseline: 42.2925x; 1.7552x over previous
import functools

import jax
import jax.numpy as jnp
from jax.experimental import pallas as pl
from jax.experimental.pallas import tpu as pltpu

K = 16
L = 2048
B = 8
RT = 256
NEG_INF = float("-inf")
POS_INF = float("inf")
HIGHEST = jax.lax.Precision.HIGHEST


def _dist_tile(qx, qy, ct_ref, n, self_col):
    kx = ct_ref[0, 0:1, :]
    ky = ct_ref[0, 1:2, :]
    dx = qx - kx
    dy = qy - ky
    d2 = dx * dx + dy * dy
    col = jax.lax.broadcasted_iota(jnp.int32, d2.shape, 1)
    d2 = jnp.where(col == self_col, d2 + 1e10, d2)
    return jnp.where(col < n, d2, POS_INF), col


def _knn_stats_body(len_ref, q_ref, ct_ref, dsum_ref, mx_ref, mn_ref):
    b = pl.program_id(0)
    j = pl.program_id(1)
    n = len_ref[b]
    q = q_ref[0]
    row = j * RT + jax.lax.broadcasted_iota(jnp.int32, (RT, L), 0)
    v, _ = _dist_tile(q[:, 0:1], q[:, 1:2], ct_ref, n, row)

    m = jnp.min(v, axis=1, keepdims=True)
    s0 = jnp.sqrt(m)
    acc = s0
    last = s0
    for _ in range(1, K):
        m = jnp.min(jnp.where(v > m, v, POS_INF), axis=1, keepdims=True)
        last = jnp.sqrt(m)
        acc = acc + last
    dsum_ref[0, 0, :] = acc[:, 0]

    rv = row[:, 0:1] < n
    t_mx = jnp.max(jnp.where(rv, last, NEG_INF))
    t_mn = jnp.min(jnp.where(rv, s0, POS_INF))

    @pl.when((b == 0) & (j == 0))
    def _():
        mx_ref[...] = jnp.full((1, 1), NEG_INF, jnp.float32)
        mn_ref[...] = jnp.full((1, 1), POS_INF, jnp.float32)

    mx_ref[...] = jnp.maximum(mx_ref[...], t_mx)
    mn_ref[...] = jnp.minimum(mn_ref[...], t_mn)


def _run_knn_stats(lengths, coords, coords_t):
    grid_spec = pltpu.PrefetchScalarGridSpec(
        num_scalar_prefetch=1,
        grid=(B, L // RT),
        in_specs=[
            pl.BlockSpec((1, RT, 2), lambda b, j, lens: (b, j, 0)),
            pl.BlockSpec((1, 2, L), lambda b, j, lens: (b, 0, 0)),
        ],
        out_specs=[
            pl.BlockSpec((1, 1, RT), lambda b, j, lens: (b, 0, j)),
            pl.BlockSpec((1, 1), lambda b, j, lens: (0, 0)),
            pl.BlockSpec((1, 1), lambda b, j, lens: (0, 0)),
        ],
    )
    return pl.pallas_call(
        _knn_stats_body,
        grid_spec=grid_spec,
        out_shape=[
            jax.ShapeDtypeStruct((B, 1, L), jnp.float32),
            jax.ShapeDtypeStruct((1, 1), jnp.float32),
            jax.ShapeDtypeStruct((1, 1), jnp.float32),
        ],
    )(lengths, coords, coords_t)


def _topk_with_idx(v, col):
    m = jnp.min(v, axis=1, keepdims=True)
    am = jnp.min(jnp.where(v == m, col, L), axis=1, keepdims=True)
    idxs, ds = [am], [jnp.sqrt(m)]
    for _ in range(1, K):
        m = jnp.min(jnp.where(v > m, v, POS_INF), axis=1, keepdims=True)
        am = jnp.min(jnp.where(v == m, col, L), axis=1, keepdims=True)
        idxs.append(am)
        ds.append(jnp.sqrt(m))
    return jnp.concatenate(idxs, axis=1), jnp.concatenate(ds, axis=1)


def _head_topk_body(len_ref, c_ref, ct_ref, hn_ref, hd_ref, nb_ref, dd_ref):
    n = len_ref[pl.program_id(0)]
    cb = c_ref[0]
    hx = cb[0:1, 0:1]
    hy = cb[0:1, 1:2]
    vh, colH = _dist_tile(hx, hy, ct_ref, n, 0)
    n0, d0 = _topk_with_idx(vh, colH)
    hn_ref[0] = n0
    hd_ref[0] = d0

    rowLK = jax.lax.broadcasted_iota(jnp.int32, (L, K), 0)
    ohT = (rowLK == n0).astype(jnp.float32)
    _gath = functools.partial(
        jax.lax.dot_general,
        dimension_numbers=(((0,), (0,)), ((), ())),
        precision=HIGHEST, preferred_element_type=jnp.float32)
    qc = _gath(ohT, cb)
    liota = jax.lax.broadcasted_iota(jnp.int32, (L, 1), 0).astype(jnp.float32)
    n0col = _gath(ohT, liota).astype(jnp.int32)
    v, colK = _dist_tile(qc[:, 0:1], qc[:, 1:2], ct_ref, n, n0col)
    nb, dd = _topk_with_idx(v, colK)
    nb_ref[0] = nb
    dd_ref[0] = dd


def _run_head_topk(lengths, coords, coords_t):
    grid_spec = pltpu.PrefetchScalarGridSpec(
        num_scalar_prefetch=1,
        grid=(B,),
        in_specs=[
            pl.BlockSpec((1, L, 2), lambda b, lens: (b, 0, 0)),
            pl.BlockSpec((1, 2, L), lambda b, lens: (b, 0, 0)),
        ],
        out_specs=[
            pl.BlockSpec((1, 1, K), lambda b, lens: (b, 0, 0)),
            pl.BlockSpec((1, 1, K), lambda b, lens: (b, 0, 0)),
            pl.BlockSpec((1, K, K), lambda b, lens: (b, 0, 0)),
            pl.BlockSpec((1, K, K), lambda b, lens: (b, 0, 0)),
        ],
    )
    return pl.pallas_call(
        _head_topk_body,
        grid_spec=grid_spec,
        out_shape=[
            jax.ShapeDtypeStruct((B, 1, K), jnp.int32),
            jax.ShapeDtypeStruct((B, 1, K), jnp.float32),
            jax.ShapeDtypeStruct((B, K, K), jnp.int32),
            jax.ShapeDtypeStruct((B, K, K), jnp.float32),
        ],
    )(lengths, coords, coords_t)


def _deg_body(len_ref, dsum_ref, mx_ref, mn_ref, dinv_ref, stats_ref):
    mxv = mx_ref[...]
    mnv = mn_ref[...]
    ir = 1.0 / (mxv - mnv)
    liota = jax.lax.broadcasted_iota(jnp.int32, (1, L), 1)
    for b in range(B):
        n = len_ref[b]
        valid = liota < n
        deg = 1.0 + (K * mxv - dsum_ref[b, 0, :][None, :]) * ir
        dinv_ref[b, 0, :] = jnp.where(valid, jax.lax.rsqrt(deg), 0.0)[0, :]
    srow = jax.lax.broadcasted_iota(jnp.int32, (8, 128), 0)
    stats_ref[...] = jnp.where(srow == 0, mxv, jnp.where(srow == 1, ir, 0.0))


def _run_deg(lengths, dsum, mx, mn):
    grid_spec = pltpu.PrefetchScalarGridSpec(
        num_scalar_prefetch=1,
        grid=(1,),
        in_specs=[
            pl.BlockSpec((B, 1, L), lambda i, lens: (0, 0, 0)),
            pl.BlockSpec((1, 1), lambda i, lens: (0, 0)),
            pl.BlockSpec((1, 1), lambda i, lens: (0, 0)),
        ],
        out_specs=[
            pl.BlockSpec((B, 1, L), lambda i, lens: (0, 0, 0)),
            pl.BlockSpec((8, 128), lambda i, lens: (0, 0)),
        ],
    )
    return pl.pallas_call(
        _deg_body,
        grid_spec=grid_spec,
        out_shape=[
            jax.ShapeDtypeStruct((B, 1, L), jnp.float32),
            jax.ShapeDtypeStruct((8, 128), jnp.float32),
        ],
    )(lengths, dsum, mx, mn)


def _head_body(hn_ref, hd_ref, nb_ref, dd_ref, dinv_ref, x_ref, stats_ref,
               w1_ref, b1_ref, w2_ref, b2_ref, fcw_ref, out_ref):
    dv = dinv_ref[0, 0, :]
    xb = x_ref[0]
    mxv = stats_ref[0:1, 0:1]
    ir = stats_ref[1:2, 0:1]
    n0 = hn_ref[0]
    d0 = hd_ref[0]
    NBi = nb_ref[0]
    DD = dd_ref[0]

    colL = jax.lax.broadcasted_iota(jnp.int32, (K, L), 1)
    rowL = jax.lax.broadcasted_iota(jnp.int32, (L, K), 0)
    ohT = (rowL == n0).astype(jnp.float32)
    _gath = functools.partial(
        jax.lax.dot_general,
        dimension_numbers=(((0,), (0,)), ((), ())),
        precision=HIGHEST, preferred_element_type=jnp.float32)
    XN = _gath(ohT, xb)
    dinv_n0 = _gath(ohT, dv[:, None])

    EW = (mxv - DD) * ir
    aggS = dinv_n0 * dinv_n0 * XN
    for j in range(K):
        ohj = (NBi[:, j:j + 1] == colL).astype(jnp.float32)
        Xj = jnp.dot(ohj, xb, precision=HIGHEST,
                     preferred_element_type=jnp.float32)
        dinv_j = jnp.dot(ohj, dv[:, None], precision=HIGHEST,
                         preferred_element_type=jnp.float32)
        coeff_j = dinv_n0 * EW[:, j:j + 1] * dinv_j
        aggS = aggS + coeff_j * Xj

    dh = dinv_ref[0, 0:1, 0:1]
    ew0 = (mxv - d0) * ir
    c0 = dh * ew0 * dinv_n0.reshape(1, K)
    xh = xb[0:1, :]
    aggH = dh * dh * xh + jnp.dot(c0, XN, precision=HIGHEST,
                                  preferred_element_type=jnp.float32)

    agg1 = jnp.concatenate([aggH, aggS], axis=0)
    h1 = jax.nn.relu(jnp.dot(agg1, w1_ref[...], precision=HIGHEST,
                             preferred_element_type=jnp.float32) + b1_ref[...])
    agg2 = dh * dh * h1[0:1, :] + jnp.dot(
        c0, h1[1:, :], precision=HIGHEST,
        preferred_element_type=jnp.float32)
    h2 = jax.nn.relu(jnp.dot(agg2, w2_ref[...], precision=HIGHEST,
                             preferred_element_type=jnp.float32) + b2_ref[...])
    out_ref[...] = jnp.sum(h2 * fcw_ref[...]).reshape(1, 1, 1)


def _run_head(hn, hd, nb, dd, dinv, x, stats, W1, b1, W2, b2, fc_w):
    return pl.pallas_call(
        _head_body,
        grid=(B,),
        in_specs=[
            pl.BlockSpec((1, 1, K), lambda b: (b, 0, 0)),
            pl.BlockSpec((1, 1, K), lambda b: (b, 0, 0)),
            pl.BlockSpec((1, K, K), lambda b: (b, 0, 0)),
            pl.BlockSpec((1, K, K), lambda b: (b, 0, 0)),
            pl.BlockSpec((1, 1, L), lambda b: (b, 0, 0)),
            pl.BlockSpec((1, L, 128), lambda b: (b, 0, 0)),
            pl.BlockSpec((8, 128), lambda b: (0, 0)),
            pl.BlockSpec((128, 256), lambda b: (0, 0)),
            pl.BlockSpec((1, 256), lambda b: (0, 0)),
            pl.BlockSpec((256, 256), lambda b: (0, 0)),
            pl.BlockSpec((1, 256), lambda b: (0, 0)),
            pl.BlockSpec((1, 256), lambda b: (0, 0)),
        ],
        out_specs=pl.BlockSpec((1, 1, 1), lambda b: (b, 0, 0)),
        out_shape=jax.ShapeDtypeStruct((B, 1, 1), jnp.float32),
    )(hn, hd, nb, dd, dinv, x, stats, W1, b1, W2, b2, fc_w)


def kernel(inputs, coords, targets, input_lengths, W1, b1, W2, b2, fc_w, fc_b):
    lengths = input_lengths.astype(jnp.int32)
    coords_t = coords.transpose(0, 2, 1)
    dsum, mx, mn = _run_knn_stats(lengths, coords, coords_t)
    hn, hd, nb, dd = _run_head_topk(lengths, coords, coords_t)
    dinv, stats = _run_deg(lengths, dsum, mx, mn)
    out = _run_head(hn, hd, nb, dd, dinv, inputs, stats,
                    W1, b1.reshape(1, -1), W2, b2.reshape(1, -1),
                    fc_w.reshape(1, -1))
    output_head = out[:, :, 0] + fc_b.reshape(1, 1)
    target_head = targets[:, 0, :]
    return output_head, target_head


# speedup vs baseline: 42.9010x; 1.0144x over previous
import functools

import jax
import jax.numpy as jnp
from jax.experimental import pallas as pl
from jax.experimental.pallas import tpu as pltpu

K = 16
L = 2048
B = 8
RT = 512
NEG_INF = float("-inf")
POS_INF = float("inf")
HIGHEST = jax.lax.Precision.HIGHEST


def _dist_tile(qx, qy, ct_ref, n, self_col):
    kx = ct_ref[0, 0:1, :]
    ky = ct_ref[0, 1:2, :]
    dx = qx - kx
    dy = qy - ky
    d2 = dx * dx + dy * dy
    col = jax.lax.broadcasted_iota(jnp.int32, d2.shape, 1)
    d2 = jnp.where(col == self_col, d2 + 1e10, d2)
    return jnp.where(col < n, d2, POS_INF), col


def _knn_stats_body(len_ref, q_ref, ct_ref, dsum_ref, mx_ref, mn_ref):
    b = pl.program_id(0)
    j = pl.program_id(1)
    n = len_ref[b]
    q = q_ref[0]
    row = j * RT + jax.lax.broadcasted_iota(jnp.int32, (RT, L), 0)
    v, _ = _dist_tile(q[:, 0:1], q[:, 1:2], ct_ref, n, row)

    m = jnp.min(v, axis=1, keepdims=True)
    s0 = jnp.sqrt(m)
    acc = s0
    last = s0
    for _ in range(1, K):
        m = jnp.min(jnp.where(v > m, v, POS_INF), axis=1, keepdims=True)
        last = jnp.sqrt(m)
        acc = acc + last
    dsum_ref[0, 0, :] = acc[:, 0]

    rv = row[:, 0:1] < n
    t_mx = jnp.max(jnp.where(rv, last, NEG_INF))
    t_mn = jnp.min(jnp.where(rv, s0, POS_INF))

    @pl.when((b == 0) & (j == 0))
    def _():
        mx_ref[...] = jnp.full((1, 1), NEG_INF, jnp.float32)
        mn_ref[...] = jnp.full((1, 1), POS_INF, jnp.float32)

    mx_ref[...] = jnp.maximum(mx_ref[...], t_mx)
    mn_ref[...] = jnp.minimum(mn_ref[...], t_mn)


def _run_knn_stats(lengths, coords, coords_t):
    grid_spec = pltpu.PrefetchScalarGridSpec(
        num_scalar_prefetch=1,
        grid=(B, L // RT),
        in_specs=[
            pl.BlockSpec((1, RT, 2), lambda b, j, lens: (b, j, 0)),
            pl.BlockSpec((1, 2, L), lambda b, j, lens: (b, 0, 0)),
        ],
        out_specs=[
            pl.BlockSpec((1, 1, RT), lambda b, j, lens: (b, 0, j)),
            pl.BlockSpec((1, 1), lambda b, j, lens: (0, 0)),
            pl.BlockSpec((1, 1), lambda b, j, lens: (0, 0)),
        ],
    )
    return pl.pallas_call(
        _knn_stats_body,
        grid_spec=grid_spec,
        out_shape=[
            jax.ShapeDtypeStruct((B, 1, L), jnp.float32),
            jax.ShapeDtypeStruct((1, 1), jnp.float32),
            jax.ShapeDtypeStruct((1, 1), jnp.float32),
        ],
    )(lengths, coords, coords_t)


def _topk_with_idx(v, col):
    m = jnp.min(v, axis=1, keepdims=True)
    am = jnp.min(jnp.where(v == m, col, L), axis=1, keepdims=True)
    idxs, ds = [am], [jnp.sqrt(m)]
    for _ in range(1, K):
        m = jnp.min(jnp.where(v > m, v, POS_INF), axis=1, keepdims=True)
        am = jnp.min(jnp.where(v == m, col, L), axis=1, keepdims=True)
        idxs.append(am)
        ds.append(jnp.sqrt(m))
    return jnp.concatenate(idxs, axis=1), jnp.concatenate(ds, axis=1)


def _head_topk_body(len_ref, c_ref, ct_ref, hn_ref, hd_ref, nb_ref, dd_ref):
    n = len_ref[pl.program_id(0)]
    cb = c_ref[0]
    hx = cb[0:1, 0:1]
    hy = cb[0:1, 1:2]
    vh, colH = _dist_tile(hx, hy, ct_ref, n, 0)
    n0, d0 = _topk_with_idx(vh, colH)
    hn_ref[0] = n0
    hd_ref[0] = d0

    rowLK = jax.lax.broadcasted_iota(jnp.int32, (L, K), 0)
    ohT = (rowLK == n0).astype(jnp.float32)
    _gath = functools.partial(
        jax.lax.dot_general,
        dimension_numbers=(((0,), (0,)), ((), ())),
        precision=HIGHEST, preferred_element_type=jnp.float32)
    qc = _gath(ohT, cb)
    liota = jax.lax.broadcasted_iota(jnp.int32, (L, 1), 0).astype(jnp.float32)
    n0col = _gath(ohT, liota).astype(jnp.int32)
    v, colK = _dist_tile(qc[:, 0:1], qc[:, 1:2], ct_ref, n, n0col)
    nb, dd = _topk_with_idx(v, colK)
    nb_ref[0] = nb
    dd_ref[0] = dd


def _run_head_topk(lengths, coords, coords_t):
    grid_spec = pltpu.PrefetchScalarGridSpec(
        num_scalar_prefetch=1,
        grid=(B,),
        in_specs=[
            pl.BlockSpec((1, L, 2), lambda b, lens: (b, 0, 0)),
            pl.BlockSpec((1, 2, L), lambda b, lens: (b, 0, 0)),
        ],
        out_specs=[
            pl.BlockSpec((1, 1, K), lambda b, lens: (b, 0, 0)),
            pl.BlockSpec((1, 1, K), lambda b, lens: (b, 0, 0)),
            pl.BlockSpec((1, K, K), lambda b, lens: (b, 0, 0)),
            pl.BlockSpec((1, K, K), lambda b, lens: (b, 0, 0)),
        ],
    )
    return pl.pallas_call(
        _head_topk_body,
        grid_spec=grid_spec,
        out_shape=[
            jax.ShapeDtypeStruct((B, 1, K), jnp.int32),
            jax.ShapeDtypeStruct((B, 1, K), jnp.float32),
            jax.ShapeDtypeStruct((B, K, K), jnp.int32),
            jax.ShapeDtypeStruct((B, K, K), jnp.float32),
        ],
    )(lengths, coords, coords_t)


def _deg_body(len_ref, dsum_ref, mx_ref, mn_ref, dinv_ref, stats_ref):
    mxv = mx_ref[...]
    mnv = mn_ref[...]
    ir = 1.0 / (mxv - mnv)
    liota = jax.lax.broadcasted_iota(jnp.int32, (1, L), 1)
    for b in range(B):
        n = len_ref[b]
        valid = liota < n
        deg = 1.0 + (K * mxv - dsum_ref[b, 0, :][None, :]) * ir
        dinv_ref[b, 0, :] = jnp.where(valid, jax.lax.rsqrt(deg), 0.0)[0, :]
    srow = jax.lax.broadcasted_iota(jnp.int32, (8, 128), 0)
    stats_ref[...] = jnp.where(srow == 0, mxv, jnp.where(srow == 1, ir, 0.0))


def _run_deg(lengths, dsum, mx, mn):
    grid_spec = pltpu.PrefetchScalarGridSpec(
        num_scalar_prefetch=1,
        grid=(1,),
        in_specs=[
            pl.BlockSpec((B, 1, L), lambda i, lens: (0, 0, 0)),
            pl.BlockSpec((1, 1), lambda i, lens: (0, 0)),
            pl.BlockSpec((1, 1), lambda i, lens: (0, 0)),
        ],
        out_specs=[
            pl.BlockSpec((B, 1, L), lambda i, lens: (0, 0, 0)),
            pl.BlockSpec((8, 128), lambda i, lens: (0, 0)),
        ],
    )
    return pl.pallas_call(
        _deg_body,
        grid_spec=grid_spec,
        out_shape=[
            jax.ShapeDtypeStruct((B, 1, L), jnp.float32),
            jax.ShapeDtypeStruct((8, 128), jnp.float32),
        ],
    )(lengths, dsum, mx, mn)


def _head_body(hn_ref, hd_ref, nb_ref, dd_ref, dinv_ref, x_ref, stats_ref,
               w1_ref, b1_ref, w2_ref, b2_ref, fcw_ref, out_ref):
    dv = dinv_ref[0, 0, :]
    xb = x_ref[0]
    mxv = stats_ref[0:1, 0:1]
    ir = stats_ref[1:2, 0:1]
    n0 = hn_ref[0]
    d0 = hd_ref[0]
    NBi = nb_ref[0]
    DD = dd_ref[0]

    colL = jax.lax.broadcasted_iota(jnp.int32, (K, L), 1)
    rowL = jax.lax.broadcasted_iota(jnp.int32, (L, K), 0)
    ohT = (rowL == n0).astype(jnp.float32)
    _gath = functools.partial(
        jax.lax.dot_general,
        dimension_numbers=(((0,), (0,)), ((), ())),
        precision=HIGHEST, preferred_element_type=jnp.float32)
    XN = _gath(ohT, xb)
    dinv_n0 = _gath(ohT, dv[:, None])

    EW = (mxv - DD) * ir
    aggS = dinv_n0 * dinv_n0 * XN
    for j in range(K):
        ohj = (NBi[:, j:j + 1] == colL).astype(jnp.float32)
        Xj = jnp.dot(ohj, xb, precision=HIGHEST,
                     preferred_element_type=jnp.float32)
        dinv_j = jnp.dot(ohj, dv[:, None], precision=HIGHEST,
                         preferred_element_type=jnp.float32)
        coeff_j = dinv_n0 * EW[:, j:j + 1] * dinv_j
        aggS = aggS + coeff_j * Xj

    dh = dinv_ref[0, 0:1, 0:1]
    ew0 = (mxv - d0) * ir
    c0 = dh * ew0 * dinv_n0.reshape(1, K)
    xh = xb[0:1, :]
    aggH = dh * dh * xh + jnp.dot(c0, XN, precision=HIGHEST,
                                  preferred_element_type=jnp.float32)

    agg1 = jnp.concatenate([aggH, aggS], axis=0)
    h1 = jax.nn.relu(jnp.dot(agg1, w1_ref[...], precision=HIGHEST,
                             preferred_element_type=jnp.float32) + b1_ref[...])
    agg2 = dh * dh * h1[0:1, :] + jnp.dot(
        c0, h1[1:, :], precision=HIGHEST,
        preferred_element_type=jnp.float32)
    h2 = jax.nn.relu(jnp.dot(agg2, w2_ref[...], precision=HIGHEST,
                             preferred_element_type=jnp.float32) + b2_ref[...])
    out_ref[...] = jnp.sum(h2 * fcw_ref[...]).reshape(1, 1, 1)


def _run_head(hn, hd, nb, dd, dinv, x, stats, W1, b1, W2, b2, fc_w):
    return pl.pallas_call(
        _head_body,
        grid=(B,),
        in_specs=[
            pl.BlockSpec((1, 1, K), lambda b: (b, 0, 0)),
            pl.BlockSpec((1, 1, K), lambda b: (b, 0, 0)),
            pl.BlockSpec((1, K, K), lambda b: (b, 0, 0)),
            pl.BlockSpec((1, K, K), lambda b: (b, 0, 0)),
            pl.BlockSpec((1, 1, L), lambda b: (b, 0, 0)),
            pl.BlockSpec((1, L, 128), lambda b: (b, 0, 0)),
            pl.BlockSpec((8, 128), lambda b: (0, 0)),
            pl.BlockSpec((128, 256), lambda b: (0, 0)),
            pl.BlockSpec((1, 256), lambda b: (0, 0)),
            pl.BlockSpec((256, 256), lambda b: (0, 0)),
            pl.BlockSpec((1, 256), lambda b: (0, 0)),
            pl.BlockSpec((1, 256), lambda b: (0, 0)),
        ],
        out_specs=pl.BlockSpec((1, 1, 1), lambda b: (b, 0, 0)),
        out_shape=jax.ShapeDtypeStruct((B, 1, 1), jnp.float32),
    )(hn, hd, nb, dd, dinv, x, stats, W1, b1, W2, b2, fc_w)


def kernel(inputs, coords, targets, input_lengths, W1, b1, W2, b2, fc_w, fc_b):
    lengths = input_lengths.astype(jnp.int32)
    coords_t = coords.transpose(0, 2, 1)
    dsum, mx, mn = _run_knn_stats(lengths, coords, coords_t)
    hn, hd, nb, dd = _run_head_topk(lengths, coords, coords_t)
    dinv, stats = _run_deg(lengths, dsum, mx, mn)
    out = _run_head(hn, hd, nb, dd, dinv, inputs, stats,
                    W1, b1.reshape(1, -1), W2, b2.reshape(1, -1),
                    fc_w.reshape(1, -1))
    output_head = out[:, :, 0] + fc_b.reshape(1, 1)
    target_head = targets[:, 0, :]
    return output_head, target_head


# speedup vs baseline: 45.8316x; 1.0683x over previous
import functools

import jax
import jax.numpy as jnp
from jax import lax
from jax.experimental import pallas as pl
from jax.experimental.pallas import tpu as pltpu
from jax.experimental.pallas import tpu_sc as plsc

K = 16
L = 2048
B = 8
RT = 512
NEG_INF = float("-inf")
POS_INF = float("inf")
HIGHEST = jax.lax.Precision.HIGHEST


def _dist_tile(qx, qy, ct_ref, n, self_col):
    kx = ct_ref[0, 0:1, :]
    ky = ct_ref[0, 1:2, :]
    dx = qx - kx
    dy = qy - ky
    d2 = dx * dx + dy * dy
    col = jax.lax.broadcasted_iota(jnp.int32, d2.shape, 1)
    d2 = jnp.where(col == self_col, d2 + 1e10, d2)
    return jnp.where(col < n, d2, POS_INF), col


def _knn_stats_body(len_ref, q_ref, ct_ref, dsum_ref, mx_ref, mn_ref):
    b = pl.program_id(0)
    j = pl.program_id(1)
    n = len_ref[b]
    q = q_ref[0]
    row = j * RT + jax.lax.broadcasted_iota(jnp.int32, (RT, L), 0)
    v, _ = _dist_tile(q[:, 0:1], q[:, 1:2], ct_ref, n, row)

    m = jnp.min(v, axis=1, keepdims=True)
    s0 = jnp.sqrt(m)
    acc = s0
    last = s0
    for _ in range(1, K):
        m = jnp.min(jnp.where(v > m, v, POS_INF), axis=1, keepdims=True)
        last = jnp.sqrt(m)
        acc = acc + last
    dsum_ref[0, 0, :] = acc[:, 0]

    rv = row[:, 0:1] < n
    t_mx = jnp.max(jnp.where(rv, last, NEG_INF))
    t_mn = jnp.min(jnp.where(rv, s0, POS_INF))

    @pl.when((b == 0) & (j == 0))
    def _():
        mx_ref[...] = jnp.full((1, 1), NEG_INF, jnp.float32)
        mn_ref[...] = jnp.full((1, 1), POS_INF, jnp.float32)

    mx_ref[...] = jnp.maximum(mx_ref[...], t_mx)
    mn_ref[...] = jnp.minimum(mn_ref[...], t_mn)


def _run_knn_stats(lengths, coords, coords_t):
    grid_spec = pltpu.PrefetchScalarGridSpec(
        num_scalar_prefetch=1,
        grid=(B, L // RT),
        in_specs=[
            pl.BlockSpec((1, RT, 2), lambda b, j, lens: (b, j, 0)),
            pl.BlockSpec((1, 2, L), lambda b, j, lens: (b, 0, 0)),
        ],
        out_specs=[
            pl.BlockSpec((1, 1, RT), lambda b, j, lens: (b, 0, j)),
            pl.BlockSpec((1, 1), lambda b, j, lens: (0, 0)),
            pl.BlockSpec((1, 1), lambda b, j, lens: (0, 0)),
        ],
    )
    return pl.pallas_call(
        _knn_stats_body,
        grid_spec=grid_spec,
        out_shape=[
            jax.ShapeDtypeStruct((B, 1, L), jnp.float32),
            jax.ShapeDtypeStruct((1, 1), jnp.float32),
            jax.ShapeDtypeStruct((1, 1), jnp.float32),
        ],
    )(lengths, coords, coords_t)


def _topk_with_idx(v, col):
    m = jnp.min(v, axis=1, keepdims=True)
    am = jnp.min(jnp.where(v == m, col, L), axis=1, keepdims=True)
    idxs, ds = [am], [jnp.sqrt(m)]
    for _ in range(1, K):
        m = jnp.min(jnp.where(v > m, v, POS_INF), axis=1, keepdims=True)
        am = jnp.min(jnp.where(v == m, col, L), axis=1, keepdims=True)
        idxs.append(am)
        ds.append(jnp.sqrt(m))
    return jnp.concatenate(idxs, axis=1), jnp.concatenate(ds, axis=1)


def _head_topk_body(len_ref, c_ref, ct_ref, hn_ref, hd_ref, nb_ref, dd_ref):
    n = len_ref[pl.program_id(0)]
    cb = c_ref[0]
    hx = cb[0:1, 0:1]
    hy = cb[0:1, 1:2]
    vh, colH = _dist_tile(hx, hy, ct_ref, n, 0)
    n0, d0 = _topk_with_idx(vh, colH)
    hn_ref[0] = n0
    hd_ref[0] = d0

    rowLK = jax.lax.broadcasted_iota(jnp.int32, (L, K), 0)
    ohT = (rowLK == n0).astype(jnp.float32)
    _gath = functools.partial(
        jax.lax.dot_general,
        dimension_numbers=(((0,), (0,)), ((), ())),
        precision=HIGHEST, preferred_element_type=jnp.float32)
    qc = _gath(ohT, cb)
    liota = jax.lax.broadcasted_iota(jnp.int32, (L, 1), 0).astype(jnp.float32)
    n0col = _gath(ohT, liota).astype(jnp.int32)
    v, colK = _dist_tile(qc[:, 0:1], qc[:, 1:2], ct_ref, n, n0col)
    nb, dd = _topk_with_idx(v, colK)
    nb_ref[0] = nb
    dd_ref[0] = dd


def _run_head_topk(lengths, coords, coords_t):
    grid_spec = pltpu.PrefetchScalarGridSpec(
        num_scalar_prefetch=1,
        grid=(B,),
        in_specs=[
            pl.BlockSpec((1, L, 2), lambda b, lens: (b, 0, 0)),
            pl.BlockSpec((1, 2, L), lambda b, lens: (b, 0, 0)),
        ],
        out_specs=[
            pl.BlockSpec((1, 1, K), lambda b, lens: (b, 0, 0)),
            pl.BlockSpec((1, 1, K), lambda b, lens: (b, 0, 0)),
            pl.BlockSpec((1, K, K), lambda b, lens: (b, 0, 0)),
            pl.BlockSpec((1, K, K), lambda b, lens: (b, 0, 0)),
        ],
    )
    return pl.pallas_call(
        _head_topk_body,
        grid_spec=grid_spec,
        out_shape=[
            jax.ShapeDtypeStruct((B, 1, K), jnp.int32),
            jax.ShapeDtypeStruct((B, 1, K), jnp.float32),
            jax.ShapeDtypeStruct((B, K, K), jnp.int32),
            jax.ShapeDtypeStruct((B, K, K), jnp.float32),
        ],
    )(lengths, coords, coords_t)


def _deg_body(len_ref, dsum_ref, mx_ref, mn_ref, dinv_ref, stats_ref):
    mxv = mx_ref[...]
    mnv = mn_ref[...]
    ir = 1.0 / (mxv - mnv)
    liota = jax.lax.broadcasted_iota(jnp.int32, (1, L), 1)
    for b in range(B):
        n = len_ref[b]
        valid = liota < n
        deg = 1.0 + (K * mxv - dsum_ref[b, 0, :][None, :]) * ir
        dinv_ref[b, 0, :] = jnp.where(valid, jax.lax.rsqrt(deg), 0.0)[0, :]
    srow = jax.lax.broadcasted_iota(jnp.int32, (8, 128), 0)
    stats_ref[...] = jnp.where(srow == 0, mxv, jnp.where(srow == 1, ir, 0.0))


def _run_deg(lengths, dsum, mx, mn):
    grid_spec = pltpu.PrefetchScalarGridSpec(
        num_scalar_prefetch=1,
        grid=(1,),
        in_specs=[
            pl.BlockSpec((B, 1, L), lambda i, lens: (0, 0, 0)),
            pl.BlockSpec((1, 1), lambda i, lens: (0, 0)),
            pl.BlockSpec((1, 1), lambda i, lens: (0, 0)),
        ],
        out_specs=[
            pl.BlockSpec((B, 1, L), lambda i, lens: (0, 0, 0)),
            pl.BlockSpec((8, 128), lambda i, lens: (0, 0)),
        ],
    )
    return pl.pallas_call(
        _deg_body,
        grid_spec=grid_spec,
        out_shape=[
            jax.ShapeDtypeStruct((B, 1, L), jnp.float32),
            jax.ShapeDtypeStruct((8, 128), jnp.float32),
        ],
    )(lengths, dsum, mx, mn)


XD = 256


def _sc_gather(xd, hnf, nbf):
    mesh = plsc.VectorSubcoreMesh(core_axis_name="c", subcore_axis_name="s")

    @functools.partial(
        pl.kernel, mesh=mesh,
        out_type=[
            jax.ShapeDtypeStruct((B, 256, XD), jnp.float32),
            jax.ShapeDtypeStruct((B, 17, XD), jnp.float32),
        ],
        scratch_types=[
            pltpu.VMEM((K,), jnp.int32),
            pltpu.VMEM((4 * K,), jnp.int32),
            pltpu.VMEM((4 * K, XD), jnp.float32),
            pltpu.VMEM((17, XD), jnp.float32),
        ],
    )
    def g(x_hbm, hn_hbm, nb_hbm, xg_out, xn_out, n0_v, nb_v, xr_v, xn_v):
        w = lax.axis_index("s") * 2 + lax.axis_index("c")
        b = w // 4
        q = w % 4
        pltpu.sync_copy(hn_hbm.at[b], n0_v)
        pltpu.sync_copy(nb_hbm.at[b, pl.ds(q * 64, 64)], nb_v)
        for i in range(4):
            nbi = nb_v[pl.ds(i * K, K)]
            pltpu.sync_copy(x_hbm.at[nbi + b * L], xr_v.at[pl.ds(i * K, K)])
        pltpu.sync_copy(xr_v, xg_out.at[b, pl.ds(q * 64, 64)])

        @pl.when(q == 0)
        def _():
            n0vals = n0_v[...]
            pltpu.sync_copy(x_hbm.at[n0vals + b * L], xn_v.at[pl.ds(0, K)])
            pltpu.sync_copy(x_hbm.at[pl.ds(b * L, 1)], xn_v.at[pl.ds(K, 1)])
            pltpu.sync_copy(xn_v, xn_out.at[b])

    return g(xd, hnf, nbf)


def _head_body(xg_ref, xn_ref, stats_ref,
               w1_ref, b1_ref, w2_ref, b2_ref, fcw_ref, out_ref):
    mxv = stats_ref[0:1, 0:1]
    ir = stats_ref[1:2, 0:1]
    XG = xg_ref[0]
    XN17 = xn_ref[0]

    rowR = jax.lax.broadcasted_iota(jnp.int32, (K * K, 17), 0)
    colR = jax.lax.broadcasted_iota(jnp.int32, (K * K, 17), 1)
    Rrep = (rowR // K == colR).astype(jnp.float32)
    Gn0 = jnp.dot(Rrep, XN17[:, 128:144], precision=HIGHEST,
                  preferred_element_type=jnp.float32)

    d2f = ((XG[:, 136:137] - Gn0[:, 8:9]) ** 2
           + (XG[:, 140:141] - Gn0[:, 12:13]) ** 2)
    df = jnp.sqrt(d2f)
    ewf = (mxv - df) * ir
    cf = Gn0[:, 0:1] * ewf * XG[:, 128:129]

    rowS = jax.lax.broadcasted_iota(jnp.int32, (K, K * K), 0)
    colS = jax.lax.broadcasted_iota(jnp.int32, (K, K * K), 1)
    S = (colS // K == rowS).astype(jnp.float32)
    dn0col = XN17[0:16, 128:129]
    dh = XN17[16:17, 128:129]
    d0col = jnp.sqrt((XN17[0:16, 136:137] - XN17[16:17, 136:137]) ** 2
                     + (XN17[0:16, 140:141] - XN17[16:17, 140:141]) ** 2)
    c0T = dh * (mxv - d0col) * ir * dn0col
    _dotg0 = functools.partial(
        jax.lax.dot_general,
        dimension_numbers=(((0,), (0,)), ((), ())),
        precision=HIGHEST, preferred_element_type=jnp.float32)

    XW = jnp.dot(XG[:, 0:128], w1_ref[...],
                 preferred_element_type=jnp.float32)
    XNW = jnp.dot(XN17[:, 0:128], w1_ref[...],
                  preferred_element_type=jnp.float32)
    h1S = jax.nn.relu(
        b1_ref[...] + dn0col * dn0col * XNW[0:16, :]
        + jnp.dot(S, cf * XW, precision=HIGHEST,
                  preferred_element_type=jnp.float32))
    h1H = jax.nn.relu(
        b1_ref[...] + dh * dh * XNW[16:17, :] + _dotg0(c0T, XNW[0:16, :]))
    h1 = jnp.concatenate([h1H, h1S], axis=0)
    HW = jnp.dot(h1, w2_ref[...],
                 preferred_element_type=jnp.float32)
    h2 = jax.nn.relu(
        b2_ref[...] + dh * dh * HW[0:1, :] + _dotg0(c0T, HW[1:, :]))
    out = jax.lax.dot_general(
        h2, fcw_ref[...], dimension_numbers=(((1,), (1,)), ((), ())),
        preferred_element_type=jnp.float32)
    out_ref[...] = out.reshape(1, 1, 1)


def _run_head(xg, xn17, stats, W1, b1, W2, b2, fc_w):
    return pl.pallas_call(
        _head_body,
        grid=(B,),
        in_specs=[
            pl.BlockSpec((1, 256, XD), lambda b: (b, 0, 0)),
            pl.BlockSpec((1, 17, XD), lambda b: (b, 0, 0)),
            pl.BlockSpec((8, 128), lambda b: (0, 0)),
            pl.BlockSpec((128, 256), lambda b: (0, 0)),
            pl.BlockSpec((1, 256), lambda b: (0, 0)),
            pl.BlockSpec((256, 256), lambda b: (0, 0)),
            pl.BlockSpec((1, 256), lambda b: (0, 0)),
            pl.BlockSpec((1, 256), lambda b: (0, 0)),
        ],
        out_specs=pl.BlockSpec((1, 1, 1), lambda b: (b, 0, 0)),
        out_shape=jax.ShapeDtypeStruct((B, 1, 1), jnp.float32),
    )(xg, xn17, stats, W1, b1, W2, b2, fc_w)


def kernel(inputs, coords, targets, input_lengths, W1, b1, W2, b2, fc_w, fc_b):
    lengths = input_lengths.astype(jnp.int32)
    coords_t = coords.transpose(0, 2, 1)
    dsum, mx, mn = _run_knn_stats(lengths, coords, coords_t)
    hn, hd, nb, dd = _run_head_topk(lengths, coords, coords_t)
    dinv, stats = _run_deg(lengths, dsum, mx, mn)
    xd = jnp.concatenate([
        inputs.reshape(B * L, 128),
        jnp.broadcast_to(dinv.reshape(B * L, 1), (B * L, 8)),
        jnp.broadcast_to(coords.reshape(B * L, 2)[:, 0:1], (B * L, 4)),
        jnp.broadcast_to(coords.reshape(B * L, 2)[:, 1:2], (B * L, 4)),
        jnp.zeros((B * L, XD - 144), jnp.float32),
    ], axis=1)
    xg, xn17 = _sc_gather(xd, hn.reshape(B, K), nb.reshape(B, K * K))
    out = _run_head(xg, xn17, stats,
                    W1, b1.reshape(1, -1), W2, b2.reshape(1, -1),
                    fc_w.reshape(1, -1))
    output_head = out[:, :, 0] + fc_b.reshape(1, 1)
    target_head = targets[:, 0, :]
    return output_head, target_head
